# baseline (speedup 1.0000x reference)
import functools

import jax
import jax.numpy as jnp
from jax import lax
from jax.experimental import pallas as pl
from jax.experimental.pallas import tpu as pltpu
from jax.experimental.pallas import tpu_sc as plsc

BATCH = 4
PREFIX_LEN = 128
NUM_VIRTUAL_TOKENS = 128
ROW_DIM = 131072
CHUNK = 2048
NCH = ROW_DIM // CHUNK
B_TOTAL = BATCH * PREFIX_LEN
LANES = 16
NDIR = 8
NB = 24
SD = 3

_info = plsc.get_sparse_core_info()
NC, NS = _info.num_cores, _info.num_subcores
CH_PER_CORE = NCH // NC
B_PER_TILE = B_TOTAL // NS
ROWS_PER_TILE = NUM_VIRTUAL_TOKENS // NS


@functools.partial(
    pl.kernel,
    out_type=jax.ShapeDtypeStruct((BATCH, PREFIX_LEN, ROW_DIM), jnp.float32),
    mesh=plsc.VectorSubcoreMesh(core_axis_name="c", subcore_axis_name="s"),
    scratch_types=[
        pltpu.VMEM((B_PER_TILE,), jnp.int32),
        pltpu.VMEM((NB, CHUNK), jnp.float32),
        pltpu.VMEM_SHARED((NUM_VIRTUAL_TOKENS, CHUNK), jnp.float32),
        pltpu.VMEM_SHARED((NUM_VIRTUAL_TOKENS, CHUNK), jnp.float32),
        pltpu.VMEM_SHARED((NUM_VIRTUAL_TOKENS, CHUNK), jnp.float32),
        pltpu.SemaphoreType.DMA,
        pltpu.SemaphoreType.DMA,
        pltpu.SemaphoreType.DMA,
        pltpu.SemaphoreType.DMA,
        pltpu.SemaphoreType.DMA,
        pltpu.SemaphoreType.DMA,
        pltpu.SemaphoreType.DMA,
        pltpu.SemaphoreType.DMA,
    ],
)
def _gather_kernel(prefix_hbm, table_hbm, out_hbm, idx_v, bb,
                   sh0, sh1, sh2, ls0, ls1, ls2, ws0, ws1, ws2,
                   bcsem, bssem):
    sid = lax.axis_index("s")
    cid = lax.axis_index("c")
    j0 = cid * CH_PER_CORE
    base_b = sid * B_PER_TILE
    bi = base_b // PREFIX_LEN
    pos0 = base_b % PREFIX_LEN
    pltpu.sync_copy(prefix_hbm.at[pl.ds(base_b, B_PER_TILE)], idx_v)
    ids = (idx_v[pl.ds(0, LANES)], idx_v[pl.ds(LANES, LANES)])
    row0 = sid * ROWS_PER_TILE
    sh, lsem, wsem = (sh0, sh1, sh2), (ls0, ls1, ls2), (ws0, ws1, ws2)

    def start_load(r, k):
        pltpu.async_copy(
            table_hbm.at[pl.ds(row0, ROWS_PER_TILE),
                         pl.ds((j0 + r) * CHUNK, CHUNK)],
            sh[k].at[pl.ds(row0, ROWS_PER_TILE)],
            lsem[k])

    def drain_load(k):
        pltpu.make_async_copy(
            table_hbm.at[pl.ds(0, ROWS_PER_TILE), pl.ds(0, CHUNK)],
            sh[k].at[pl.ds(row0, ROWS_PER_TILE)],
            lsem[k]).wait()

    def fire_writes(r, k, first=False):
        for i in range(NDIR):
            v = ids[i // LANES][i % LANES]
            pltpu.async_copy(
                sh[k].at[v],
                out_hbm.at[bi, pos0 + i, pl.ds((j0 + r) * CHUNK, CHUNK)],
                wsem[k])
        if not first:
            pltpu.make_async_copy(
                table_hbm.at[pl.ds(0, NB), pl.ds(0, CHUNK)],
                bb, bssem).wait()
        for i in range(NB):
            p = NDIR + i
            v = ids[p // LANES][p % LANES]
            pltpu.async_copy(sh[k].at[v], bb.at[i], bcsem)
        pltpu.make_async_copy(
            table_hbm.at[pl.ds(0, NB), pl.ds(0, CHUNK)],
            bb, bcsem).wait()
        pltpu.async_copy(
            bb,
            out_hbm.at[bi, pl.ds(pos0 + NDIR, NB),
                       pl.ds((j0 + r) * CHUNK, CHUNK)],
            bssem)

    def drain_writes(k):
        pltpu.make_async_copy(
            table_hbm.at[pl.ds(0, NDIR), pl.ds(0, CHUNK)],
            out_hbm.at[bi, pl.ds(pos0, NDIR), pl.ds(0, CHUNK)],
            wsem[k]).wait()

    start_load(0, 0)
    start_load(1, 1)
    start_load(2, 2)
    drain_load(0)
    plsc.subcore_barrier()
    fire_writes(0, 0, first=True)

    def body(g, carry):
        for k0 in range(SD):
            r = SD * g + k0 + 1
            ka = (k0 + 1) % SD
            drain_load(ka)
            plsc.subcore_barrier()
            fire_writes(r, ka)
            drain_writes(k0)
            plsc.subcore_barrier()
            @pl.when(r + 2 < CH_PER_CORE)
            def _():
                start_load(r + 2, k0)
        return carry

    lax.fori_loop(0, (CH_PER_CORE - 2) // SD, body, 0)
    drain_load(1)
    plsc.subcore_barrier()
    fire_writes(CH_PER_CORE - 1, 1)
    drain_writes(0)
    drain_writes(1)
    pltpu.make_async_copy(
        table_hbm.at[pl.ds(0, NB), pl.ds(0, CHUNK)],
        bb, bssem).wait()


def kernel(prefix, table):
    return _gather_kernel(prefix.reshape(B_TOTAL), table)


# speedup vs baseline: 1.2694x; 1.2694x over previous
import functools

import jax
import jax.numpy as jnp
from jax import lax
from jax.experimental import pallas as pl
from jax.experimental.pallas import tpu as pltpu
from jax.experimental.pallas import tpu_sc as plsc

BATCH = 4
PREFIX_LEN = 128
NUM_VIRTUAL_TOKENS = 128
ROW_DIM = 131072
CHUNK = 2048
NCH = ROW_DIM // CHUNK
B_TOTAL = BATCH * PREFIX_LEN
LANES = 16
NDIR = 24
NB = 8
SD = 3

_info = plsc.get_sparse_core_info()
NC, NS = _info.num_cores, _info.num_subcores
CH_PER_CORE = NCH // NC
B_PER_TILE = B_TOTAL // NS
ROWS_PER_TILE = NUM_VIRTUAL_TOKENS // NS


@functools.partial(
    pl.kernel,
    out_type=jax.ShapeDtypeStruct((BATCH, PREFIX_LEN, ROW_DIM), jnp.float32),
    mesh=plsc.VectorSubcoreMesh(core_axis_name="c", subcore_axis_name="s"),
    scratch_types=[
        pltpu.VMEM((B_PER_TILE,), jnp.int32),
        pltpu.VMEM((NB, CHUNK), jnp.float32),
        pltpu.VMEM_SHARED((NUM_VIRTUAL_TOKENS, CHUNK), jnp.float32),
        pltpu.VMEM_SHARED((NUM_VIRTUAL_TOKENS, CHUNK), jnp.float32),
        pltpu.VMEM_SHARED((NUM_VIRTUAL_TOKENS, CHUNK), jnp.float32),
        pltpu.SemaphoreType.DMA,
        pltpu.SemaphoreType.DMA,
        pltpu.SemaphoreType.DMA,
        pltpu.SemaphoreType.DMA,
        pltpu.SemaphoreType.DMA,
        pltpu.SemaphoreType.DMA,
        pltpu.SemaphoreType.DMA,
        pltpu.SemaphoreType.DMA,
    ],
)
def _gather_kernel(prefix_hbm, table_hbm, out_hbm, idx_v, bb,
                   sh0, sh1, sh2, ls0, ls1, ls2, ws0, ws1, ws2,
                   bcsem, bssem):
    sid = lax.axis_index("s")
    cid = lax.axis_index("c")
    j0 = cid * CH_PER_CORE
    base_b = sid * B_PER_TILE
    bi = base_b // PREFIX_LEN
    pos0 = base_b % PREFIX_LEN
    pltpu.sync_copy(prefix_hbm.at[pl.ds(base_b, B_PER_TILE)], idx_v)
    ids = (idx_v[pl.ds(0, LANES)], idx_v[pl.ds(LANES, LANES)])
    row0 = sid * ROWS_PER_TILE
    sh, lsem, wsem = (sh0, sh1, sh2), (ls0, ls1, ls2), (ws0, ws1, ws2)

    def start_load(r, k):
        pltpu.async_copy(
            table_hbm.at[pl.ds(row0, ROWS_PER_TILE),
                         pl.ds((j0 + r) * CHUNK, CHUNK)],
            sh[k].at[pl.ds(row0, ROWS_PER_TILE)],
            lsem[k])

    def drain_load(k):
        pltpu.make_async_copy(
            table_hbm.at[pl.ds(0, ROWS_PER_TILE), pl.ds(0, CHUNK)],
            sh[k].at[pl.ds(row0, ROWS_PER_TILE)],
            lsem[k]).wait()

    def fire_writes(r, k, first=False):
        for i in range(NDIR):
            v = ids[i // LANES][i % LANES]
            pltpu.async_copy(
                sh[k].at[v],
                out_hbm.at[bi, pos0 + i, pl.ds((j0 + r) * CHUNK, CHUNK)],
                wsem[k])
        if not first:
            pltpu.make_async_copy(
                table_hbm.at[pl.ds(0, NB), pl.ds(0, CHUNK)],
                bb, bssem).wait()
        for i in range(NB):
            p = NDIR + i
            v = ids[p // LANES][p % LANES]
            pltpu.async_copy(sh[k].at[v], bb.at[i], bcsem)
        pltpu.make_async_copy(
            table_hbm.at[pl.ds(0, NB), pl.ds(0, CHUNK)],
            bb, bcsem).wait()
        pltpu.async_copy(
            bb,
            out_hbm.at[bi, pl.ds(pos0 + NDIR, NB),
                       pl.ds((j0 + r) * CHUNK, CHUNK)],
            bssem)

    def drain_writes(k):
        pltpu.make_async_copy(
            table_hbm.at[pl.ds(0, NDIR), pl.ds(0, CHUNK)],
            out_hbm.at[bi, pl.ds(pos0, NDIR), pl.ds(0, CHUNK)],
            wsem[k]).wait()

    start_load(0, 0)
    start_load(1, 1)
    start_load(2, 2)
    drain_load(0)
    plsc.subcore_barrier()
    fire_writes(0, 0, first=True)

    def body(g, carry):
        for k0 in range(SD):
            r = SD * g + k0 + 1
            ka = (k0 + 1) % SD
            drain_load(ka)
            plsc.subcore_barrier()
            fire_writes(r, ka)
            drain_writes(k0)
            plsc.subcore_barrier()
            @pl.when(r + 2 < CH_PER_CORE)
            def _():
                start_load(r + 2, k0)
        return carry

    lax.fori_loop(0, (CH_PER_CORE - 2) // SD, body, 0)
    drain_load(1)
    plsc.subcore_barrier()
    fire_writes(CH_PER_CORE - 1, 1)
    drain_writes(0)
    drain_writes(1)
    pltpu.make_async_copy(
        table_hbm.at[pl.ds(0, NB), pl.ds(0, CHUNK)],
        bb, bssem).wait()


def kernel(prefix, table):
    return _gather_kernel(prefix.reshape(B_TOTAL), table)
